# baseline (device time: 26371 ns/iter reference)
import jax
import jax.numpy as jnp
from jax import lax
from jax.experimental import pallas as pl
from jax.experimental.pallas import tpu as pltpu

N_DEV = 32
NX, NY, NZ = 2, 4, 4

_PERM8 = tuple(
    yp * 2 + (xg if yp % 2 == 0 else 1 - xg)
    for xg in range(NX)
    for yp in range(NY)
)


def kernel(x, w_mat):
    m, k_per = x.shape
    _, n = w_mat.shape
    m_per = m // N_DEV
    m_blk = m // NZ

    def body(x_ref, w_ref, out_ref,
             partial, xrecv, axbuf, yrecv, ymrecv, bzbuf, zrecv, sbuf, srecv,
             xsend_sems, xrecv_sems, ysend_sems, yrecv_sems,
             ymsend_sems, ymrecv_sems,
             zsend_sems, zrecv_sems, ssend_sem, srecv_sem):
        me = lax.axis_index("i")
        z = me // 8
        r = me % 8
        yy = r // 2
        j = r % 2
        xx = jnp.where(yy % 2 == 0, j, 1 - j)

        def y_peer(k):
            yp = (yy + 1 + k) % NY
            jp = jnp.where(yp % 2 == 0, xx, 1 - xx)
            return yp, z * 8 + yp * 2 + jp

        barrier_sem = pltpu.get_barrier_semaphore()
        pl.semaphore_signal(
            barrier_sem, inc=1,
            device_id=(me ^ 1,), device_id_type=pl.DeviceIdType.MESH,
        )
        for k in range(NY - 1):
            _, tgt = y_peer(k)
            pl.semaphore_signal(
                barrier_sem, inc=1,
                device_id=(tgt,), device_id_type=pl.DeviceIdType.MESH,
            )
        for k in range(NZ - 1):
            zp = (z + 1 + k) % NZ
            pl.semaphore_signal(
                barrier_sem, inc=1,
                device_id=(zp * 8 + r,), device_id_type=pl.DeviceIdType.MESH,
            )

        qs = [(z + p) % NZ for p in range(NZ)]

        q0 = qs[0]
        p_q = jnp.dot(
            x_ref[pl.ds(q0 * m_blk, m_blk), :], w_ref[:, :],
            preferred_element_type=jnp.float32,
        )
        chunks = p_q.astype(jnp.bfloat16).reshape(NX * NY, m_per, n)
        partial[0] = jnp.stack(
            [chunks[t] for t in _PERM8]
        ).reshape(NX, NY, m_per, n)

        pl.semaphore_wait(barrier_sem, 7)

        y0_rdmas = []
        for k in range(NY - 1):
            yp, tgt = y_peer(k)
            rd = pltpu.make_async_remote_copy(
                src_ref=partial.at[0, xx, yp],
                dst_ref=yrecv.at[0, k],
                send_sem=ysend_sems.at[0, k],
                recv_sem=yrecv_sems.at[0, k],
                device_id=(tgt,),
                device_id_type=pl.DeviceIdType.MESH,
            )
            rd.start()
            y0_rdmas.append(rd)
            rdm = pltpu.make_async_remote_copy(
                src_ref=partial.at[0, 1 - xx, yp],
                dst_ref=ymrecv.at[k],
                send_sem=ymsend_sems.at[k],
                recv_sem=ymrecv_sems.at[k],
                device_id=(tgt,),
                device_id_type=pl.DeviceIdType.MESH,
            )
            rdm.start()
            y0_rdmas.append(rdm)

        x_rdmas = []
        for p in range(1, NZ):
            q = qs[p]
            p_q = jnp.dot(
                x_ref[pl.ds(q * m_blk, m_blk), :], w_ref[:, :],
                preferred_element_type=jnp.float32,
            )
            chunks = p_q.astype(jnp.bfloat16).reshape(NX * NY, m_per, n)
            partial[p] = jnp.stack(
                [chunks[t] for t in _PERM8]
            ).reshape(NX, NY, m_per, n)

            rd = pltpu.make_async_remote_copy(
                src_ref=partial.at[p, 1 - xx],
                dst_ref=xrecv.at[p - 1],
                send_sem=xsend_sems.at[p - 1],
                recv_sem=xrecv_sems.at[p - 1],
                device_id=(me ^ 1,),
                device_id_type=pl.DeviceIdType.MESH,
            )
            rd.start()
            x_rdmas.append(rd)

        s_mine = partial[0, xx, yy].astype(jnp.float32)
        s_mirror = partial[0, 1 - xx, yy].astype(jnp.float32)
        for k in range(NY - 1):
            y0_rdmas[2 * k].wait_recv()
            s_mine = s_mine + yrecv[0, k].astype(jnp.float32)
            y0_rdmas[2 * k + 1].wait_recv()
            s_mirror = s_mirror + ymrecv[k].astype(jnp.float32)
        sbuf[:, :] = s_mirror.astype(jnp.bfloat16)
        s_rdma = pltpu.make_async_remote_copy(
            src_ref=sbuf,
            dst_ref=srecv,
            send_sem=ssend_sem.at[0],
            recv_sem=srecv_sem.at[0],
            device_id=(me ^ 1,),
            device_id_type=pl.DeviceIdType.MESH,
        )
        s_rdma.start()

        y_rdmas = []
        for p in range(1, NZ):
            x_rdmas[p - 1].wait_recv()
            axbuf[p - 1] = partial[p, xx] + xrecv[p - 1]
            for k in range(NY - 1):
                yp, tgt = y_peer(k)
                rd = pltpu.make_async_remote_copy(
                    src_ref=axbuf.at[p - 1, yp],
                    dst_ref=yrecv.at[p, k],
                    send_sem=ysend_sems.at[p, k],
                    recv_sem=yrecv_sems.at[p, k],
                    device_id=(tgt,),
                    device_id_type=pl.DeviceIdType.MESH,
                )
                rd.start()
                y_rdmas.append(rd)

        z_rdmas = []
        for p in range(1, NZ):
            q = qs[p]
            acc_q = (partial[p, xx, yy].astype(jnp.float32)
                     + xrecv[p - 1, yy].astype(jnp.float32))
            for k in range(NY - 1):
                y_rdmas[(p - 1) * (NY - 1) + k].wait_recv()
                acc_q = acc_q + yrecv[p, k].astype(jnp.float32)
            bzbuf[p - 1] = acc_q.astype(jnp.bfloat16)
            rd = pltpu.make_async_remote_copy(
                src_ref=bzbuf.at[p - 1],
                dst_ref=zrecv.at[p - 1],
                send_sem=zsend_sems.at[p - 1],
                recv_sem=zrecv_sems.at[p - 1],
                device_id=(q * 8 + r,),
                device_id_type=pl.DeviceIdType.MESH,
            )
            rd.start()
            z_rdmas.append(rd)

        s_rdma.wait_recv()
        final_acc = s_mine + srecv[:, :].astype(jnp.float32)
        for k in range(NZ - 1):
            z_rdmas[k].wait_recv()
            final_acc = final_acc + zrecv[k].astype(jnp.float32)

        c = 0.7978845608028654
        out_ref[:, :] = 0.5 * final_acc * (
            1.0 + jnp.tanh(c * (final_acc
                                + 0.044715 * final_acc * final_acc * final_acc))
        )

        for rd in x_rdmas + y0_rdmas + y_rdmas + z_rdmas + [s_rdma]:
            rd.wait_send()

    return pl.pallas_call(
        body,
        out_shape=jax.ShapeDtypeStruct((m_per, n), jnp.float32),
        in_specs=[
            pl.BlockSpec(memory_space=pltpu.VMEM),
            pl.BlockSpec(memory_space=pltpu.VMEM),
        ],
        out_specs=pl.BlockSpec(memory_space=pltpu.VMEM),
        scratch_shapes=[
            pltpu.VMEM((NZ, NX, NY, m_per, n), jnp.bfloat16),
            pltpu.VMEM((NZ - 1, NY, m_per, n), jnp.bfloat16),
            pltpu.VMEM((NZ - 1, NY, m_per, n), jnp.bfloat16),
            pltpu.VMEM((NZ, NY - 1, m_per, n), jnp.bfloat16),
            pltpu.VMEM((NY - 1, m_per, n), jnp.bfloat16),
            pltpu.VMEM((NZ - 1, m_per, n), jnp.bfloat16),
            pltpu.VMEM((NZ - 1, m_per, n), jnp.bfloat16),
            pltpu.VMEM((m_per, n), jnp.bfloat16),
            pltpu.VMEM((m_per, n), jnp.bfloat16),
            pltpu.SemaphoreType.DMA((NZ - 1,)),
            pltpu.SemaphoreType.DMA((NZ - 1,)),
            pltpu.SemaphoreType.DMA((NZ, NY - 1)),
            pltpu.SemaphoreType.DMA((NZ, NY - 1)),
            pltpu.SemaphoreType.DMA((NY - 1,)),
            pltpu.SemaphoreType.DMA((NY - 1,)),
            pltpu.SemaphoreType.DMA((NZ - 1,)),
            pltpu.SemaphoreType.DMA((NZ - 1,)),
            pltpu.SemaphoreType.DMA((1,)),
            pltpu.SemaphoreType.DMA((1,)),
        ],
        compiler_params=pltpu.CompilerParams(collective_id=0),
    )(x, w_mat)


# device time: 25593 ns/iter; 1.0304x vs baseline; 1.0304x over previous
import jax
import jax.numpy as jnp
from jax import lax
from jax.experimental import pallas as pl
from jax.experimental.pallas import tpu as pltpu

N_DEV = 32
NX, NY, NZ = 2, 4, 4

_PERM8 = tuple(
    yp * 2 + (xg if yp % 2 == 0 else 1 - xg)
    for xg in range(NX)
    for yp in range(NY)
)


def kernel(x, w_mat):
    m, k_per = x.shape
    _, n = w_mat.shape
    m_per = m // N_DEV
    m_blk = m // NZ

    def body(x_ref, w_ref, out_ref,
             partial, xrecv, axbuf, yrecv, bzbuf, zrecv,
             xsend_sems, xrecv_sems, ysend_sems, yrecv_sems,
             zsend_sems, zrecv_sems):
        me = lax.axis_index("i")
        z = me // 8
        r = me % 8
        yy = r // 2
        j = r % 2
        xx = jnp.where(yy % 2 == 0, j, 1 - j)

        qs = [(z + 1 + kq) % NZ if kq < NZ - 1 else z for kq in range(NZ)]

        barrier_sem = pltpu.get_barrier_semaphore()
        pl.semaphore_signal(
            barrier_sem, inc=1,
            device_id=(me ^ 1,), device_id_type=pl.DeviceIdType.MESH,
        )
        for k in range(NY - 1):
            yp = (yy + 1 + k) % NY
            jp = jnp.where(yp % 2 == 0, xx, 1 - xx)
            pl.semaphore_signal(
                barrier_sem, inc=1,
                device_id=(z * 8 + yp * 2 + jp,),
                device_id_type=pl.DeviceIdType.MESH,
            )
        for k in range(NZ - 1):
            zp = (z + 1 + k) % NZ
            pl.semaphore_signal(
                barrier_sem, inc=1,
                device_id=(zp * 8 + r,),
                device_id_type=pl.DeviceIdType.MESH,
            )

        x_rdmas = []
        for kq in range(NZ):
            q = qs[kq]
            p_q = jnp.dot(
                x_ref[pl.ds(q * m_blk, m_blk), :], w_ref[:, :],
                preferred_element_type=jnp.float32,
            )
            chunks = p_q.astype(jnp.bfloat16).reshape(NX * NY, m_per, n)
            partial[kq] = jnp.stack(
                [chunks[t] for t in _PERM8]
            ).reshape(NX, NY, m_per, n)

            if kq == 0:
                pl.semaphore_wait(barrier_sem, 7)
            rd = pltpu.make_async_remote_copy(
                src_ref=partial.at[kq, 1 - xx],
                dst_ref=xrecv.at[kq],
                send_sem=xsend_sems.at[kq],
                recv_sem=xrecv_sems.at[kq],
                device_id=(me ^ 1,),
                device_id_type=pl.DeviceIdType.MESH,
            )
            rd.start()
            x_rdmas.append(rd)

        y_rdmas = []
        for kq in range(NZ):
            x_rdmas[kq].wait_recv()
            axbuf[kq] = partial[kq, xx] + xrecv[kq]

            for k in range(NY - 1):
                yp = (yy + 1 + k) % NY
                jp = jnp.where(yp % 2 == 0, xx, 1 - xx)
                tgt = z * 8 + yp * 2 + jp
                rd = pltpu.make_async_remote_copy(
                    src_ref=axbuf.at[kq, yp],
                    dst_ref=yrecv.at[kq, k],
                    send_sem=ysend_sems.at[kq, k],
                    recv_sem=yrecv_sems.at[kq, k],
                    device_id=(tgt,),
                    device_id_type=pl.DeviceIdType.MESH,
                )
                rd.start()
                y_rdmas.append(rd)

        z_rdmas = []
        final_acc = None
        for kq in range(NZ):
            q = qs[kq]
            acc_q = (partial[kq, xx, yy].astype(jnp.float32)
                     + xrecv[kq, yy].astype(jnp.float32))
            for k in range(NY - 1):
                y_rdmas[kq * (NY - 1) + k].wait_recv()
                acc_q = acc_q + yrecv[kq, k].astype(jnp.float32)

            if kq < NZ - 1:
                bzbuf[kq] = acc_q.astype(jnp.bfloat16)
                rd = pltpu.make_async_remote_copy(
                    src_ref=bzbuf.at[kq],
                    dst_ref=zrecv.at[kq],
                    send_sem=zsend_sems.at[kq],
                    recv_sem=zrecv_sems.at[kq],
                    device_id=(q * 8 + r,),
                    device_id_type=pl.DeviceIdType.MESH,
                )
                rd.start()
                z_rdmas.append(rd)
            else:
                final_acc = acc_q

        for k in range(NZ - 1):
            z_rdmas[k].wait_recv()
            final_acc = final_acc + zrecv[k].astype(jnp.float32)

        c = 0.7978845608028654
        out_ref[:, :] = 0.5 * final_acc * (
            1.0 + jnp.tanh(c * (final_acc
                                + 0.044715 * final_acc * final_acc * final_acc))
        )

        for rd in x_rdmas + y_rdmas + z_rdmas:
            rd.wait_send()

    return pl.pallas_call(
        body,
        out_shape=jax.ShapeDtypeStruct((m_per, n), jnp.float32),
        in_specs=[
            pl.BlockSpec(memory_space=pltpu.VMEM),
            pl.BlockSpec(memory_space=pltpu.VMEM),
        ],
        out_specs=pl.BlockSpec(memory_space=pltpu.VMEM),
        scratch_shapes=[
            pltpu.VMEM((NZ, NX, NY, m_per, n), jnp.bfloat16),
            pltpu.VMEM((NZ, NY, m_per, n), jnp.bfloat16),
            pltpu.VMEM((NZ, NY, m_per, n), jnp.bfloat16),
            pltpu.VMEM((NZ, NY - 1, m_per, n), jnp.bfloat16),
            pltpu.VMEM((NZ - 1, m_per, n), jnp.bfloat16),
            pltpu.VMEM((NZ - 1, m_per, n), jnp.bfloat16),
            pltpu.SemaphoreType.DMA((NZ,)),
            pltpu.SemaphoreType.DMA((NZ,)),
            pltpu.SemaphoreType.DMA((NZ, NY - 1)),
            pltpu.SemaphoreType.DMA((NZ, NY - 1)),
            pltpu.SemaphoreType.DMA((NZ - 1,)),
            pltpu.SemaphoreType.DMA((NZ - 1,)),
        ],
        compiler_params=pltpu.CompilerParams(collective_id=0),
    )(x, w_mat)
